# baseline (speedup 1.0000x reference)
import functools

import jax
import jax.numpy as jnp
from jax import lax
from jax.experimental import pallas as pl
from jax.experimental.pallas import tpu as pltpu
from jax.experimental.pallas import tpu_sc as plsc

_V = 262144
_GRID = (64, 64, 64)
_C = 16
_CH = 8
_N = 2000000
_L = 16
_NR = 64
_VPR = _V // _NR
_BLK = 2048
_GRP = _BLK // _L
_W = 128
_BR = _BLK // _W
_NROW = _N // _W
_VROW = _VPR // _W
_AROW = _VPR // _W


def _read_scalar(vref, pos):
    return vref[pl.ds(pos, _L)][0]


def _sc_body(feats, idx_hbm, starts_hbm, out, starts_v, idx_v, feat_v, acc,
             cnt, sem):
    sid = lax.axis_index("s")
    w = sid * 2 + lax.axis_index("c")
    pltpu.sync_copy(starts_hbm, starts_v)
    zeros = jnp.zeros((_L,), jnp.float32)
    ones = jnp.ones((_L,), jnp.float32)
    lane = lax.iota(jnp.int32, _L)

    def _issue(prow, b, buf, co):
        rowoff = pl.multiple_of(prow + b * _BR, 8)
        pltpu.async_copy(idx_hbm.at[pl.ds(rowoff, _BR)],
                         idx_v.at[pl.ds(buf * _BR, _BR)], sem)
        for c in range(_CH):
            pltpu.async_copy(
                feats.at[co + c, pl.ds(rowoff, _BR)],
                feat_v.at[pl.ds((buf * _CH + c) * _BR, _BR)], sem)

    def _drain(buf):
        pltpu.make_async_copy(idx_hbm.at[pl.ds(0, _BR)],
                              idx_v.at[pl.ds(buf * _BR, _BR)], sem).wait()
        for c in range(_CH):
            pltpu.make_async_copy(
                feats.at[0, pl.ds(0, _BR)],
                feat_v.at[pl.ds((buf * _CH + c) * _BR, _BR)], sem).wait()

    for rr in range(2):
        for h in range(2):
            r = w * 2 + rr
            co = h * _CH
            vbase = r * _VPR
            p0 = _read_scalar(starts_v, r)
            p1 = _read_scalar(starts_v, r + 1)

            def _zero(i, carry):
                for k in range(_W // _L):
                    s = pl.ds(k * _L, _L)
                    cnt[i, s] = zeros
                    for c in range(_CH):
                        acc[c * _AROW + i, s] = zeros
                return carry

            lax.fori_loop(0, _AROW, _zero, 0)

            pa = (p0 // 1024) * 1024
            prow = pl.multiple_of(pa // _W, 8)
            nblk = (p1 - pa + _BLK - 1) // _BLK
            npair = jnp.maximum((nblk + 1) // 2, 1)

            def _process(b, buf):
                off = pa + b * _BLK
                lo = jnp.maximum(p0, off)
                hi = jnp.minimum(p1, off + _BLK)

                def _one_group(j):
                    brow = j // (_W // _L)
                    bcol = (j % (_W // _L)) * _L
                    g = off + j * _L + lane
                    iv = idx_v[buf * _BR + brow, pl.ds(bcol, _L)]
                    lidx = iv - vbase
                    m = ((g >= lo) & (g < hi)
                         & (lidx >= 0) & (lidx < _VPR))
                    row = lax.shift_right_logical(lidx, 7)
                    col = lax.bitwise_and(lidx, _W - 1)
                    plsc.addupdate_scatter(cnt, [row, col], ones, mask=m)
                    for c in range(_CH):
                        fv = feat_v[(buf * _CH + c) * _BR + brow,
                                    pl.ds(bcol, _L)]
                        plsc.addupdate_scatter(acc, [row + (c * _AROW), col],
                                               fv, mask=m)

                def _group(j, carry2):
                    _one_group(2 * j)
                    _one_group(2 * j + 1)
                    return carry2

                lax.fori_loop(0, _GRP // 2, _group, 0)

            def _pair(i, carry):
                b = 2 * i
                _issue(prow, b + 1, 1, co)
                _drain(0)
                _process(b, 0)
                _issue(prow, b + 2, 0, co)
                _drain(1)
                _process(b + 1, 1)
                return carry

            _issue(prow, 0, 0, co)
            lax.fori_loop(0, npair, _pair, 0)
            _drain(0)

            def _mean(i, carry):
                for k in range(_W // _L):
                    s = pl.ds(k * _L, _L)
                    rcp = 1.0 / jnp.maximum(cnt[i, s], 1.0)
                    for c in range(_CH):
                        acc[c * _AROW + i, s] = acc[c * _AROW + i, s] * rcp
                return carry

            lax.fori_loop(0, _AROW, _mean, 0)
            for c in range(_CH):
                pltpu.sync_copy(
                    acc.at[pl.ds(c * _AROW, _AROW)],
                    out.at[pl.ds((co + c) * (_V // _W) + r * _VROW, _VROW)])


_mesh = plsc.VectorSubcoreMesh(core_axis_name="c", subcore_axis_name="s")

_voxelize = functools.partial(
    pl.kernel,
    mesh=_mesh,
    out_type=jax.ShapeDtypeStruct((_C * _V // _W, _W), jnp.float32),
    compiler_params=pltpu.CompilerParams(needs_layout_passes=False),
    scratch_types=[
        pltpu.VMEM((96,), jnp.int32),
        pltpu.VMEM((2 * _BR, _W), jnp.int32),
        pltpu.VMEM((2 * _CH * _BR, _W), jnp.float32),
        pltpu.VMEM((_CH * _AROW, _W), jnp.float32),
        pltpu.VMEM((_AROW, _W), jnp.float32),
        pltpu.SemaphoreType.DMA,
    ],
)(_sc_body)


@jax.jit
def kernel(features, indices):
    feats2d = features.reshape(_C, _NROW, _W)
    idx = indices.astype(jnp.int32).reshape(_NROW, _W)
    bounds = jnp.arange(_NR, dtype=jnp.int32) * _VPR
    starts = jnp.searchsorted(indices.astype(jnp.int32), bounds,
                              side="left").astype(jnp.int32)
    starts = jnp.concatenate([starts, jnp.full((32,), _N, jnp.int32)])
    out = _voxelize(feats2d, idx, starts)
    return out.reshape((1, _C) + _GRID)
